# Initial kernel scaffold; baseline (speedup 1.0000x reference)
import jax
import jax.numpy as jnp
from jax.experimental import pallas as pl


def kernel(x, edge_index, edge_attr, query_time, params):
    raise NotImplementedError("write your pallas kernel here")


# speedup vs baseline: 1.3250x; 1.3250x over previous
import functools

import jax
import jax.numpy as jnp
from jax import lax
from jax.experimental import pallas as pl
from jax.experimental.pallas import tpu as pltpu
from jax.experimental.pallas import tpu_sc as plsc

_EPS = 1e-7
_NS = 16
_NC = 2
_CH = 128


def _enc_body(x_ref, w_ref, b_ref, o_ref):
    h = jnp.maximum(
        jnp.dot(x_ref[...], w_ref[...], preferred_element_type=jnp.float32)
        + b_ref[...], 0.0)
    hh = h.shape[1] // 2
    o_ref[0] = h[:, :hh]
    o_ref[1] = h[:, hh:]


def _encode(x, W, b):
    N, Din = x.shape
    bn = N // 5
    H = W.shape[1]
    return pl.pallas_call(
        _enc_body,
        grid=(N // bn,),
        in_specs=[
            pl.BlockSpec((bn, Din), lambda i: (i, 0)),
            pl.BlockSpec((Din, H), lambda i: (0, 0)),
            pl.BlockSpec((1, H), lambda i: (0, 0)),
        ],
        out_specs=pl.BlockSpec((2, bn, H // 2), lambda i: (0, i, 0)),
        out_shape=jax.ShapeDtypeStruct((2, N, H // 2), jnp.float32),
    )(x, W, b.reshape(1, H))


def _edge_body(a_ref, w1_ref, b1_ref, w2_ref, b2_ref, o_ref):
    t = jnp.maximum(
        jnp.dot(a_ref[...], w1_ref[...], preferred_element_type=jnp.float32)
        + b1_ref[...], 0.0)
    z = jnp.dot(t, w2_ref[...], preferred_element_type=jnp.float32) + b2_ref[...]
    o_ref[...] = 1.0 / (1.0 + jnp.exp(-z))


def _edge_weights(ea, W1, b1, W2, b2):
    E, De = ea.shape
    be = E // 160
    H = W1.shape[1]
    return pl.pallas_call(
        _edge_body,
        grid=(E // be,),
        in_specs=[
            pl.BlockSpec((be, De), lambda i: (i, 0)),
            pl.BlockSpec((De, H), lambda i: (0, 0)),
            pl.BlockSpec((1, H), lambda i: (0, 0)),
            pl.BlockSpec((H, 1), lambda i: (0, 0)),
            pl.BlockSpec((1, 1), lambda i: (0, 0)),
        ],
        out_specs=pl.BlockSpec((be, 1), lambda i: (i, 0)),
        out_shape=jax.ShapeDtypeStruct((E, 1), jnp.float32),
    )(ea, W1, b1.reshape(1, H), W2, b2.reshape(1, 1))


def _sc_aggregate(hsplit, src, dst, ew, wle, blee, compute_deg):
    TN, Hh = hsplit.shape
    Nn = TN // 2
    E = src.shape[0]
    nchunks = E // _CH
    kmax = (nchunks + _NS - 1) // _NS
    spacing = (Nn // (_NS * 8)) * 8
    extra = Nn - _NS * spacing
    dpt = Nn // 5
    nvec = Hh // 16

    mesh = plsc.VectorSubcoreMesh(
        core_axis_name="c", subcore_axis_name="s",
        num_cores=_NC, num_subcores=_NS)

    out_type = [jax.ShapeDtypeStruct((TN, Hh), jnp.float32)]
    if compute_deg:
        out_type.append(jax.ShapeDtypeStruct((Nn,), jnp.float32))

    scratch = [
        pltpu.VMEM((_CH,), jnp.int32),
        pltpu.VMEM((_CH,), jnp.int32),
        pltpu.VMEM((_CH,), jnp.int32),
        pltpu.VMEM((_CH,), jnp.float32),
        pltpu.VMEM((_CH, Hh), jnp.float32),
        pltpu.VMEM((_CH, Hh), jnp.float32),
        pltpu.VMEM((Hh,), jnp.float32),
        pltpu.VMEM((Hh,), jnp.float32),
        pltpu.VMEM((_CH,), jnp.float32),
        pltpu.VMEM((dpt,), jnp.float32),
        pltpu.VMEM_SHARED((Nn, Hh), jnp.float32),
        pltpu.VMEM_SHARED((Nn,), jnp.float32),
        pltpu.SemaphoreType.DMA,
    ]

    @functools.partial(pl.kernel, out_type=out_type, mesh=mesh,
                       scratch_types=scratch)
    def body(h_r, src_r, dst_r, ew_r, wle_r, blee_r, *rest):
        if compute_deg:
            agg_o, deg_o = rest[0], rest[1]
            (src_v, gidx_v, dst_v, ew_v, gbuf, msg_v, wle_v, blee_v, ones_v,
             zbuf, agg_sh, deg_sh, gsem) = rest[2:]
        else:
            agg_o = rest[0]
            (src_v, gidx_v, dst_v, ew_v, gbuf, msg_v, wle_v, blee_v, ones_v,
             zbuf, agg_sh, deg_sh, gsem) = rest[1:]
        c = lax.axis_index("c")
        s = lax.axis_index("s")
        zeros16 = jnp.zeros((16,), jnp.float32)
        ones16 = jnp.full((16,), 1.0, jnp.float32)
        epsv = jnp.full((16,), _EPS, jnp.float32)

        pltpu.sync_copy(wle_r.at[pl.ds(c * Hh, Hh)], wle_v)
        pltpu.sync_copy(blee_r.at[pl.ds(c * Hh, Hh)], blee_v)

        def zrow(e, _):
            for j in range(nvec):
                msg_v[e, pl.ds(j * 16, 16)] = zeros16
            return 0
        lax.fori_loop(0, _CH, zrow, 0)
        off = 0
        while off < spacing:
            step = min(_CH, spacing - off)
            pltpu.sync_copy(msg_v.at[pl.ds(0, step)],
                            agg_sh.at[pl.ds(s * spacing + off, step)])
            off += step
        if extra:
            @pl.when(s == _NS - 1)
            def _zx():
                pltpu.sync_copy(msg_v.at[pl.ds(0, extra)],
                                agg_sh.at[pl.ds(_NS * spacing, extra)])
        if compute_deg:
            def zo(i, _):
                ones_v[pl.ds(i * 16, 16)] = ones16
                return 0
            lax.fori_loop(0, _CH // 16, zo, 0)

            @pl.when((c == 0) & (s < 5))
            def _zd():
                def zb(i, _):
                    zbuf[pl.ds(i * 16, 16)] = zeros16
                    return 0
                lax.fori_loop(0, dpt // 16, zb, 0)
                pltpu.sync_copy(zbuf, deg_sh.at[pl.ds(s * dpt, dpt)])

        plsc.subcore_barrier()

        def chunk_body(k, _):
            cid = k * _NS + s

            @pl.when(cid < nchunks)
            def _go():
                base = cid * _CH
                pltpu.sync_copy(src_r.at[pl.ds(base, _CH)], src_v)
                pltpu.sync_copy(dst_r.at[pl.ds(base, _CH)], dst_v)
                pltpu.sync_copy(ew_r.at[pl.ds(base, _CH)], ew_v)
                off = c * Nn
                for i in range(_CH // 16):
                    sl = pl.ds(i * 16, 16)
                    gidx_v[sl] = src_v[sl] + off
                pltpu.async_copy(h_r.at[gidx_v], gbuf, gsem).wait()

                def edge_group(g, _2):
                    ew16 = ew_v[pl.ds(g * 16, 16)]
                    for i in range(16):
                        e = g * 16 + i
                        w = ew16[i]
                        for j in range(nvec):
                            sl = pl.ds(j * 16, 16)
                            cj = wle_v[sl] * w + blee_v[sl]
                            msg_v[e, sl] = jnp.maximum(gbuf[e, sl] + cj, epsv)
                    return 0
                lax.fori_loop(0, _CH // 16, edge_group, 0)

                pltpu.sync_copy(msg_v, agg_sh.at[dst_v], add=True)
                if compute_deg:
                    @pl.when(c == 0)
                    def _dg():
                        pltpu.sync_copy(ones_v, deg_sh.at[dst_v], add=True)
            return 0
        lax.fori_loop(0, kmax, chunk_body, 0)

        plsc.subcore_barrier()

        pltpu.sync_copy(agg_sh.at[pl.ds(s * spacing, spacing)],
                        agg_o.at[pl.ds(c * Nn + s * spacing, spacing)])
        if extra:
            @pl.when(s == _NS - 1)
            def _wx():
                pltpu.sync_copy(agg_sh.at[pl.ds(_NS * spacing, extra)],
                                agg_o.at[pl.ds(c * Nn + _NS * spacing, extra)])
        if compute_deg:
            @pl.when((c == 0) & (s < 5))
            def _wd():
                pltpu.sync_copy(deg_sh.at[pl.ds(s * dpt, dpt)], zbuf)
                pltpu.sync_copy(zbuf, deg_o.at[pl.ds(s * dpt, dpt)])

    return body(hsplit, src, dst, ew, wle, blee)


def _mlp_body(agg_ref, h_ref, deg_ref, w1_ref, b1_ref, w2_ref, b2_ref, o_ref):
    hb = jnp.concatenate([h_ref[0], h_ref[1]], axis=1)
    ab = jnp.concatenate([agg_ref[0], agg_ref[1]], axis=1)
    deg = jnp.maximum(deg_ref[...], 1.0)
    out = ab / deg + hb
    t = jnp.maximum(
        jnp.dot(out, w1_ref[...], preferred_element_type=jnp.float32)
        + b1_ref[...], 0.0)
    hn = jnp.dot(t, w2_ref[...], preferred_element_type=jnp.float32) + b2_ref[...]
    hnew = jnp.maximum(hb + hn, 0.0)
    hh = hnew.shape[1] // 2
    o_ref[0] = hnew[:, :hh]
    o_ref[1] = hnew[:, hh:]


def _mlp(agg, h, deg, W1, b1, W2, b2):
    _, N, Hh = h.shape
    bn = N // 5
    H = 2 * Hh
    H2 = W1.shape[1]
    return pl.pallas_call(
        _mlp_body,
        grid=(N // bn,),
        in_specs=[
            pl.BlockSpec((2, bn, Hh), lambda i: (0, i, 0)),
            pl.BlockSpec((2, bn, Hh), lambda i: (0, i, 0)),
            pl.BlockSpec((bn, 1), lambda i: (i, 0)),
            pl.BlockSpec((H, H2), lambda i: (0, 0)),
            pl.BlockSpec((1, H2), lambda i: (0, 0)),
            pl.BlockSpec((H2, H), lambda i: (0, 0)),
            pl.BlockSpec((1, H), lambda i: (0, 0)),
        ],
        out_specs=pl.BlockSpec((2, bn, Hh), lambda i: (0, i, 0)),
        out_shape=jax.ShapeDtypeStruct((2, N, Hh), jnp.float32),
    )(agg, h, deg, W1, b1.reshape(1, H2), W2, b2.reshape(1, H))


def _pool_body(h_ref, o_ref):
    @pl.when(pl.program_id(0) == 0)
    def _init():
        o_ref[...] = jnp.zeros_like(o_ref)
    s0 = jnp.sum(h_ref[0], axis=0, keepdims=True)
    s1 = jnp.sum(h_ref[1], axis=0, keepdims=True)
    o_ref[...] += jnp.concatenate([s0, s1], axis=0)


def _pool(h):
    _, N, Hh = h.shape
    bn = N // 5
    return pl.pallas_call(
        _pool_body,
        grid=(N // bn,),
        in_specs=[pl.BlockSpec((2, bn, Hh), lambda i: (0, i, 0))],
        out_specs=pl.BlockSpec((2, Hh), lambda i: (0, 0)),
        out_shape=jax.ShapeDtypeStruct((2, Hh), jnp.float32),
    )(h)


def _dec_body(n_nodes, sums_ref, qt_ref, wt_ref, bt_ref, w1_ref, b1_ref,
              w2_ref, b2_ref, o_ref):
    pooled = jnp.concatenate([sums_ref[0:1, :], sums_ref[1:2, :]], axis=1)
    pooled = pooled * (1.0 / n_nodes)
    temb = jnp.maximum(
        jnp.dot(qt_ref[...], wt_ref[...], preferred_element_type=jnp.float32)
        + bt_ref[...], 0.0)
    comb = pooled + temb
    d = jnp.maximum(
        jnp.dot(comb, w1_ref[...], preferred_element_type=jnp.float32)
        + b1_ref[...], 0.0)
    o_ref[...] = jnp.dot(d, w2_ref[...], preferred_element_type=jnp.float32) \
        + b2_ref[...]


def _decoder(sums, qt, Wt, bt, W1, b1, W2, b2, n_nodes):
    B = qt.shape[0]
    H = Wt.shape[1]
    OUT = W2.shape[1]
    return pl.pallas_call(
        functools.partial(_dec_body, float(n_nodes)),
        out_shape=jax.ShapeDtypeStruct((B, OUT), jnp.float32),
    )(sums, qt, Wt, bt.reshape(1, H), W1, b1.reshape(1, H), W2,
      b2.reshape(1, OUT))


def kernel(x, edge_index, edge_attr, query_time, params):
    p = params
    N = x.shape[0]
    H = p["W_enc"].shape[1]
    Hh = H // 2
    E = edge_index.shape[1]
    L = p["Wm1"].shape[0]
    s_bn = 1.0 / jnp.sqrt(jnp.float32(1.0 + 1e-5))

    src = edge_index[0].astype(jnp.int32)
    dst = edge_index[1].astype(jnp.int32)

    hs = _encode(x, p["W_enc"], p["b_enc"])
    ew = _edge_weights(edge_attr, p["We1"], p["be1"], p["We2"],
                       p["be2"]).reshape(E)

    deg = None
    for l in range(L):
        wle = p["Wle"][l].reshape(H)
        blee = p["ble"][l] + _EPS
        res = _sc_aggregate(hs.reshape(2 * N, Hh), src, dst, ew, wle, blee,
                            compute_deg=(l == 0))
        if l == 0:
            agg, deg = res[0], res[1]
        else:
            agg = res[0]
        g1 = p["gamma"][l] * s_bn
        W1 = p["Wm1"][l] * g1[None, :]
        b1 = p["bm1"][l] * g1 + p["beta"][l]
        hs = _mlp(agg.reshape(2, N, Hh), hs, deg.reshape(N, 1), W1, b1,
                  p["Wm2"][l], p["bm2"][l])

    sums = _pool(hs)
    return _decoder(sums, query_time, p["Wt"], p["bt"], p["Wd1"], p["bd1"],
                    p["Wd2"], p["bd2"], N)


# speedup vs baseline: 3.0130x; 2.2739x over previous
import functools

import jax
import jax.numpy as jnp
from jax import lax
from jax.experimental import pallas as pl
from jax.experimental.pallas import tpu as pltpu
from jax.experimental.pallas import tpu_sc as plsc

_EPS = 1e-7
_NS = 16
_NC = 2
_CH = 64


def _enc_body(x_ref, w_ref, b_ref, o_ref):
    h = jnp.maximum(
        jnp.dot(x_ref[...], w_ref[...], preferred_element_type=jnp.float32)
        + b_ref[...], 0.0)
    hh = h.shape[1] // 2
    o_ref[0] = h[:, :hh]
    o_ref[1] = h[:, hh:]


def _encode(x, W, b):
    N, Din = x.shape
    bn = N // 5
    H = W.shape[1]
    return pl.pallas_call(
        _enc_body,
        grid=(N // bn,),
        in_specs=[
            pl.BlockSpec((bn, Din), lambda i: (i, 0)),
            pl.BlockSpec((Din, H), lambda i: (0, 0)),
            pl.BlockSpec((1, H), lambda i: (0, 0)),
        ],
        out_specs=pl.BlockSpec((2, bn, H // 2), lambda i: (0, i, 0)),
        out_shape=jax.ShapeDtypeStruct((2, N, H // 2), jnp.float32),
    )(x, W, b.reshape(1, H))


def _edge_body(a_ref, w1_ref, b1_ref, w2_ref, b2_ref, o_ref):
    t = jnp.maximum(
        jnp.dot(a_ref[...], w1_ref[...], preferred_element_type=jnp.float32)
        + b1_ref[...], 0.0)
    z = jnp.dot(t, w2_ref[...], preferred_element_type=jnp.float32) + b2_ref[...]
    o_ref[...] = 1.0 / (1.0 + jnp.exp(-z))


def _edge_weights(ea, W1, b1, W2, b2):
    E, De = ea.shape
    be = E // 160
    H = W1.shape[1]
    return pl.pallas_call(
        _edge_body,
        grid=(E // be,),
        in_specs=[
            pl.BlockSpec((be, De), lambda i: (i, 0)),
            pl.BlockSpec((De, H), lambda i: (0, 0)),
            pl.BlockSpec((1, H), lambda i: (0, 0)),
            pl.BlockSpec((H, 1), lambda i: (0, 0)),
            pl.BlockSpec((1, 1), lambda i: (0, 0)),
        ],
        out_specs=pl.BlockSpec((be, 1), lambda i: (i, 0)),
        out_shape=jax.ShapeDtypeStruct((E, 1), jnp.float32),
    )(ea, W1, b1.reshape(1, H), W2, b2.reshape(1, 1))


_SEG = 16

def _sc_aggregate(hsplit, src2, dst2, ew2, wle, blee, nch_real, compute_deg):
    TN, Hh = hsplit.shape
    Nn = TN // 2
    npad = src2.shape[0]
    nch = npad
    tpc = npad // _NS
    nseg = tpc // _SEG
    spacing = (Nn // (_NS * 8)) * 8
    extra = Nn - _NS * spacing
    dpt = Nn // 5
    nvec = Hh // 16

    mesh = plsc.VectorSubcoreMesh(
        core_axis_name="c", subcore_axis_name="s",
        num_cores=_NC, num_subcores=_NS)

    out_type = [jax.ShapeDtypeStruct((TN, Hh), jnp.float32)]
    if compute_deg:
        out_type.append(jax.ShapeDtypeStruct((Nn,), jnp.float32))

    scratch = [
        pltpu.VMEM((_SEG, _CH), jnp.int32),
        pltpu.VMEM((2, _SEG, _CH), jnp.int32),
        pltpu.VMEM((_SEG, _CH), jnp.float32),
        pltpu.VMEM((_SEG, _CH), jnp.int32),
        pltpu.VMEM((2, _CH, Hh), jnp.float32),
        pltpu.VMEM((2, _CH, Hh), jnp.float32),
        pltpu.VMEM((Hh,), jnp.float32),
        pltpu.VMEM((Hh,), jnp.float32),
        pltpu.VMEM((_CH,), jnp.float32),
        pltpu.VMEM((dpt,), jnp.float32),
        pltpu.VMEM_SHARED((Nn, Hh), jnp.float32),
        pltpu.VMEM_SHARED((Nn,), jnp.float32),
        pltpu.SemaphoreType.DMA,
        pltpu.SemaphoreType.DMA,
        pltpu.SemaphoreType.DMA,
        pltpu.SemaphoreType.DMA,
    ]

    @functools.partial(pl.kernel, out_type=out_type, mesh=mesh,
                       scratch_types=scratch)
    def body(h_r, src_r, dst_r, ew_r, wle_r, blee_r, *rest):
        if compute_deg:
            agg_o, deg_o = rest[0], rest[1]
            sc = rest[2:]
        else:
            agg_o = rest[0]
            sc = rest[1:]
        (src_sv, dst_sv, ew_sv, gidx_all, gbuf2, msg2, wle_v, blee_v, ones_v,
         zbuf, agg_sh, deg_sh, gsemA, gsemB, ssemA, ssemB) = sc
        c = lax.axis_index("c")
        s = lax.axis_index("s")
        zeros16 = jnp.zeros((16,), jnp.float32)
        ones16 = jnp.full((16,), 1.0, jnp.float32)
        epsv = jnp.full((16,), _EPS, jnp.float32)
        ach = jnp.minimum(tpc, jnp.maximum(0, nch_real - s * tpc))
        tile0 = s * tpc

        pltpu.sync_copy(wle_r.at[pl.ds(c * Hh, Hh)], wle_v)
        pltpu.sync_copy(blee_r.at[pl.ds(c * Hh, Hh)], blee_v)
        wle_c = [wle_v[pl.ds(j * 16, 16)] for j in range(nvec)]
        blee_c = [blee_v[pl.ds(j * 16, 16)] for j in range(nvec)]

        def zrow(e, _):
            for j in range(nvec):
                msg2[0, e, pl.ds(j * 16, 16)] = zeros16
            return 0
        lax.fori_loop(0, _CH, zrow, 0)
        off = 0
        while off < spacing:
            step = min(_CH, spacing - off)
            pltpu.sync_copy(msg2.at[0].at[pl.ds(0, step)],
                            agg_sh.at[pl.ds(s * spacing + off, step)])
            off += step
        if extra:
            @pl.when(s == _NS - 1)
            def _zx():
                pltpu.sync_copy(msg2.at[0].at[pl.ds(0, extra)],
                                agg_sh.at[pl.ds(_NS * spacing, extra)])
        if compute_deg:
            def zo(i, _):
                ones_v[pl.ds(i * 16, 16)] = ones16
                return 0
            lax.fori_loop(0, _CH // 16, zo, 0)

            @pl.when((c == 0) & (s < 5))
            def _zd():
                def zb(i, _):
                    zbuf[pl.ds(i * 16, 16)] = zeros16
                    return 0
                lax.fori_loop(0, dpt // 16, zb, 0)
                pltpu.sync_copy(zbuf, deg_sh.at[pl.ds(s * dpt, dpt)])

        plsc.subcore_barrier()

        hdummy = h_r.at[pl.ds(0, _CH)]

        def compute_msg(par, prow):
            def grp(g, _):
                ew16 = ew_sv[prow, pl.ds(g * 16, 16)]
                for i in range(16):
                    w = ew16[i]
                    for j in range(nvec):
                        sl = pl.ds(j * 16, 16)
                        msg2[par, g * 16 + i, sl] = jnp.maximum(
                            gbuf2[par, g * 16 + i, sl]
                            + (wle_c[j] * w + blee_c[j]), epsv)
                return 0
            lax.fori_loop(0, _CH // 16, grp, 0)

        def fire_gather(par, row, gsem):
            pltpu.async_copy(h_r.at[gidx_all.at[row]], gbuf2.at[par], gsem)

        def seg_body(q, _):
            segrow = tile0 + q * _SEG
            pltpu.sync_copy(src_r.at[pl.ds(segrow, _SEG)], src_sv)
            qp = q % 2
            pltpu.sync_copy(dst_r.at[pl.ds(segrow, _SEG)], dst_sv.at[qp])
            pltpu.sync_copy(ew_r.at[pl.ds(segrow, _SEG)], ew_sv)
            goff = c * Nn

            def gx(i, _):
                for j in range(_CH // 16):
                    sl = pl.ds(j * 16, 16)
                    gidx_all[i, sl] = src_sv[i, sl] + goff
                return 0
            lax.fori_loop(0, _SEG, gx, 0)

            rel0 = q * _SEG


            def pair_body(pp, _):
                half = [(0, gsemA, ssemA), (1, gsemB, ssemB)]
                for par, gsem, ssem in half:
                    prow = 2 * pp + par
                    rel = rel0 + prow
                    gpair = q * (_SEG // 2) + pp

                    @pl.when(rel < ach)
                    def _do(par=par, gsem=gsem, ssem=ssem,
                            prow=prow, rel=rel, gpair=gpair):
                        pltpu.async_copy(h_r.at[gidx_all.at[prow]],
                                         gbuf2.at[par], gsem).wait()
                        compute_msg(par, prow)
                        pltpu.async_copy(msg2.at[par],
                                         agg_sh.at[dst_sv.at[qp, prow]],
                                         ssem, add=True).wait()
                        if compute_deg:
                            @pl.when(c == 0)
                            def _dg():
                                pltpu.sync_copy(
                                    ones_v, deg_sh.at[dst_sv.at[qp, prow]],
                                    add=True)
                return 0
            lax.fori_loop(0, _SEG // 2, pair_body, 0)
            return 0
        lax.fori_loop(0, nseg, seg_body, 0)


        plsc.subcore_barrier()

        pltpu.sync_copy(agg_sh.at[pl.ds(s * spacing, spacing)],
                        agg_o.at[pl.ds(c * Nn + s * spacing, spacing)])
        if extra:
            @pl.when(s == _NS - 1)
            def _wx():
                pltpu.sync_copy(agg_sh.at[pl.ds(_NS * spacing, extra)],
                                agg_o.at[pl.ds(c * Nn + _NS * spacing, extra)])
        if compute_deg:
            @pl.when((c == 0) & (s < 5))
            def _wd():
                pltpu.sync_copy(deg_sh.at[pl.ds(s * dpt, dpt)], zbuf)
                pltpu.sync_copy(zbuf, deg_o.at[pl.ds(s * dpt, dpt)])

    return body(hsplit, src2, dst2, ew2, wle, blee)


def _mlp_body(agg_ref, h_ref, deg_ref, w1_ref, b1_ref, w2_ref, b2_ref, o_ref):
    hb = jnp.concatenate([h_ref[0], h_ref[1]], axis=1)
    ab = jnp.concatenate([agg_ref[0], agg_ref[1]], axis=1)
    deg = jnp.maximum(deg_ref[...], 1.0)
    out = ab / deg + hb
    t = jnp.maximum(
        jnp.dot(out, w1_ref[...], preferred_element_type=jnp.float32)
        + b1_ref[...], 0.0)
    hn = jnp.dot(t, w2_ref[...], preferred_element_type=jnp.float32) + b2_ref[...]
    hnew = jnp.maximum(hb + hn, 0.0)
    hh = hnew.shape[1] // 2
    o_ref[0] = hnew[:, :hh]
    o_ref[1] = hnew[:, hh:]


def _mlp(agg, h, deg, W1, b1, W2, b2):
    _, N, Hh = h.shape
    bn = N // 5
    H = 2 * Hh
    H2 = W1.shape[1]
    return pl.pallas_call(
        _mlp_body,
        grid=(N // bn,),
        in_specs=[
            pl.BlockSpec((2, bn, Hh), lambda i: (0, i, 0)),
            pl.BlockSpec((2, bn, Hh), lambda i: (0, i, 0)),
            pl.BlockSpec((bn, 1), lambda i: (i, 0)),
            pl.BlockSpec((H, H2), lambda i: (0, 0)),
            pl.BlockSpec((1, H2), lambda i: (0, 0)),
            pl.BlockSpec((H2, H), lambda i: (0, 0)),
            pl.BlockSpec((1, H), lambda i: (0, 0)),
        ],
        out_specs=pl.BlockSpec((2, bn, Hh), lambda i: (0, i, 0)),
        out_shape=jax.ShapeDtypeStruct((2, N, Hh), jnp.float32),
    )(agg, h, deg, W1, b1.reshape(1, H2), W2, b2.reshape(1, H))


def _pool_body(h_ref, o_ref):
    @pl.when(pl.program_id(0) == 0)
    def _init():
        o_ref[...] = jnp.zeros_like(o_ref)
    s0 = jnp.sum(h_ref[0], axis=0, keepdims=True)
    s1 = jnp.sum(h_ref[1], axis=0, keepdims=True)
    o_ref[...] += jnp.concatenate([s0, s1], axis=0)


def _pool(h):
    _, N, Hh = h.shape
    bn = N // 5
    return pl.pallas_call(
        _pool_body,
        grid=(N // bn,),
        in_specs=[pl.BlockSpec((2, bn, Hh), lambda i: (0, i, 0))],
        out_specs=pl.BlockSpec((2, Hh), lambda i: (0, 0)),
        out_shape=jax.ShapeDtypeStruct((2, Hh), jnp.float32),
    )(h)


def _dec_body(n_nodes, sums_ref, qt_ref, wt_ref, bt_ref, w1_ref, b1_ref,
              w2_ref, b2_ref, o_ref):
    pooled = jnp.concatenate([sums_ref[0:1, :], sums_ref[1:2, :]], axis=1)
    pooled = pooled * (1.0 / n_nodes)
    temb = jnp.maximum(
        jnp.dot(qt_ref[...], wt_ref[...], preferred_element_type=jnp.float32)
        + bt_ref[...], 0.0)
    comb = pooled + temb
    d = jnp.maximum(
        jnp.dot(comb, w1_ref[...], preferred_element_type=jnp.float32)
        + b1_ref[...], 0.0)
    o_ref[...] = jnp.dot(d, w2_ref[...], preferred_element_type=jnp.float32) \
        + b2_ref[...]


def _decoder(sums, qt, Wt, bt, W1, b1, W2, b2, n_nodes):
    B = qt.shape[0]
    H = Wt.shape[1]
    OUT = W2.shape[1]
    return pl.pallas_call(
        functools.partial(_dec_body, float(n_nodes)),
        out_shape=jax.ShapeDtypeStruct((B, OUT), jnp.float32),
    )(sums, qt, Wt, bt.reshape(1, H), W1, b1.reshape(1, H), W2,
      b2.reshape(1, OUT))


def kernel(x, edge_index, edge_attr, query_time, params):
    p = params
    N = x.shape[0]
    H = p["W_enc"].shape[1]
    Hh = H // 2
    E = edge_index.shape[1]
    L = p["Wm1"].shape[0]
    s_bn = 1.0 / jnp.sqrt(jnp.float32(1.0 + 1e-5))

    src = edge_index[0].astype(jnp.int32)
    dst = edge_index[1].astype(jnp.int32)

    hs = _encode(x, p["W_enc"], p["b_enc"])
    ew = _edge_weights(edge_attr, p["We1"], p["be1"], p["We2"],
                       p["be2"]).reshape(E)

    nch = E // _CH
    tpc = -(-nch // _NS)
    tpc = -(-tpc // _SEG) * _SEG
    npad = _NS * tpc
    src2 = jnp.pad(src.reshape(nch, _CH), ((0, npad - nch), (0, 0)))
    dst2 = jnp.pad(dst.reshape(nch, _CH), ((0, npad - nch), (0, 0)))
    ew2 = jnp.pad(ew.reshape(nch, _CH), ((0, npad - nch), (0, 0)))

    deg = None
    for l in range(L):
        wle = p["Wle"][l].reshape(H)
        blee = p["ble"][l] + _EPS
        res = _sc_aggregate(hs.reshape(2 * N, Hh), src2, dst2, ew2, wle, blee,
                            nch, compute_deg=(l == 0))
        if l == 0:
            agg, deg = res[0], res[1]
        else:
            agg = res[0]
        g1 = p["gamma"][l] * s_bn
        W1 = p["Wm1"][l] * g1[None, :]
        b1 = p["bm1"][l] * g1 + p["beta"][l]
        hs = _mlp(agg.reshape(2, N, Hh), hs, deg.reshape(N, 1), W1, b1,
                  p["Wm2"][l], p["bm2"][l])

    sums = _pool(hs)
    return _decoder(sums, query_time, p["Wt"], p["bt"], p["Wd1"], p["bd1"],
                    p["Wd2"], p["bd2"], N)


# speedup vs baseline: 3.5169x; 1.1672x over previous
import functools

import jax
import jax.numpy as jnp
from jax import lax
from jax.experimental import pallas as pl
from jax.experimental.pallas import tpu as pltpu
from jax.experimental.pallas import tpu_sc as plsc

_EPS = 1e-7
_NS = 16
_NC = 2
_CH = 64


def _enc_body(x_ref, w_ref, b_ref, o_ref):
    h = jnp.maximum(
        jnp.dot(x_ref[...], w_ref[...], preferred_element_type=jnp.float32)
        + b_ref[...], 0.0)
    hh = h.shape[1] // 2
    o_ref[0] = h[:, :hh]
    o_ref[1] = h[:, hh:]


def _encode(x, W, b):
    N, Din = x.shape
    bn = N // 5
    H = W.shape[1]
    return pl.pallas_call(
        _enc_body,
        grid=(N // bn,),
        in_specs=[
            pl.BlockSpec((bn, Din), lambda i: (i, 0)),
            pl.BlockSpec((Din, H), lambda i: (0, 0)),
            pl.BlockSpec((1, H), lambda i: (0, 0)),
        ],
        out_specs=pl.BlockSpec((2, bn, H // 2), lambda i: (0, i, 0)),
        out_shape=jax.ShapeDtypeStruct((2, N, H // 2), jnp.float32),
    )(x, W, b.reshape(1, H))


def _edge_body(a_ref, w1_ref, b1_ref, w2_ref, b2_ref, o_ref):
    t = jnp.maximum(
        jnp.dot(a_ref[...], w1_ref[...], preferred_element_type=jnp.float32)
        + b1_ref[...], 0.0)
    z = jnp.dot(t, w2_ref[...], preferred_element_type=jnp.float32) + b2_ref[...]
    o_ref[...] = 1.0 / (1.0 + jnp.exp(-z))


def _edge_weights(ea, W1, b1, W2, b2):
    E, De = ea.shape
    be = E // 160
    H = W1.shape[1]
    return pl.pallas_call(
        _edge_body,
        grid=(E // be,),
        in_specs=[
            pl.BlockSpec((be, De), lambda i: (i, 0)),
            pl.BlockSpec((De, H), lambda i: (0, 0)),
            pl.BlockSpec((1, H), lambda i: (0, 0)),
            pl.BlockSpec((H, 1), lambda i: (0, 0)),
            pl.BlockSpec((1, 1), lambda i: (0, 0)),
        ],
        out_specs=pl.BlockSpec((be, 1), lambda i: (i, 0)),
        out_shape=jax.ShapeDtypeStruct((E, 1), jnp.float32),
    )(ea, W1, b1.reshape(1, H), W2, b2.reshape(1, 1))


_SEG = 16

def _sc_aggregate(hsplit, src2, dst2, ew2, wle, blee, nch_real, compute_deg):
    TN, Hh = hsplit.shape
    Nn = TN // 2
    npad = src2.shape[0]
    nch = npad
    assert nch_real % 2 == 0
    tpc = npad // _NS
    nseg = tpc // _SEG
    spacing = (Nn // (_NS * 8)) * 8
    extra = Nn - _NS * spacing
    dpt = Nn // 5
    nvec = Hh // 16

    mesh = plsc.VectorSubcoreMesh(
        core_axis_name="c", subcore_axis_name="s",
        num_cores=_NC, num_subcores=_NS)

    out_type = [jax.ShapeDtypeStruct((TN, Hh), jnp.float32)]
    if compute_deg:
        out_type.append(jax.ShapeDtypeStruct((Nn,), jnp.float32))

    scratch = [
        pltpu.VMEM((_SEG, _CH), jnp.int32),
        pltpu.VMEM((2, _SEG, _CH), jnp.int32),
        pltpu.VMEM((_SEG, _CH), jnp.float32),
        pltpu.VMEM((_SEG, _CH), jnp.int32),
        pltpu.VMEM((2, _CH, Hh), jnp.float32),
        pltpu.VMEM((2, _CH, Hh), jnp.float32),
        pltpu.VMEM((Hh,), jnp.float32),
        pltpu.VMEM((Hh,), jnp.float32),
        pltpu.VMEM((_CH,), jnp.float32),
        pltpu.VMEM((dpt,), jnp.float32),
        pltpu.VMEM_SHARED((Nn, Hh), jnp.float32),
        pltpu.VMEM_SHARED((Nn,), jnp.float32),
        pltpu.SemaphoreType.DMA,
        pltpu.SemaphoreType.DMA,
        pltpu.SemaphoreType.DMA,
        pltpu.SemaphoreType.DMA,
    ]

    @functools.partial(pl.kernel, out_type=out_type, mesh=mesh,
                       scratch_types=scratch)
    def body(h_r, src_r, dst_r, ew_r, wle_r, blee_r, *rest):
        if compute_deg:
            agg_o, deg_o = rest[0], rest[1]
            sc = rest[2:]
        else:
            agg_o = rest[0]
            sc = rest[1:]
        (src_sv, dst_sv, ew_sv, gidx_all, gbuf2, msg2, wle_v, blee_v, ones_v,
         zbuf, agg_sh, deg_sh, gsemA, gsemB, ssemA, ssemB) = sc
        c = lax.axis_index("c")
        s = lax.axis_index("s")
        zeros16 = jnp.zeros((16,), jnp.float32)
        ones16 = jnp.full((16,), 1.0, jnp.float32)
        epsv = jnp.full((16,), _EPS, jnp.float32)
        ach = jnp.minimum(tpc, jnp.maximum(0, nch_real - s * tpc))
        tile0 = s * tpc

        pltpu.sync_copy(wle_r.at[pl.ds(c * Hh, Hh)], wle_v)
        pltpu.sync_copy(blee_r.at[pl.ds(c * Hh, Hh)], blee_v)
        wle_c = [wle_v[pl.ds(j * 16, 16)] for j in range(nvec)]
        blee_c = [blee_v[pl.ds(j * 16, 16)] for j in range(nvec)]

        def zrow(e, _):
            for j in range(nvec):
                msg2[0, e, pl.ds(j * 16, 16)] = zeros16
            return 0
        lax.fori_loop(0, _CH, zrow, 0)
        off = 0
        while off < spacing:
            step = min(_CH, spacing - off)
            pltpu.sync_copy(msg2.at[0].at[pl.ds(0, step)],
                            agg_sh.at[pl.ds(s * spacing + off, step)])
            off += step
        if extra:
            @pl.when(s == _NS - 1)
            def _zx():
                pltpu.sync_copy(msg2.at[0].at[pl.ds(0, extra)],
                                agg_sh.at[pl.ds(_NS * spacing, extra)])
        if compute_deg:
            def zo(i, _):
                ones_v[pl.ds(i * 16, 16)] = ones16
                return 0
            lax.fori_loop(0, _CH // 16, zo, 0)

            @pl.when((c == 0) & (s < 5))
            def _zd():
                def zb(i, _):
                    zbuf[pl.ds(i * 16, 16)] = zeros16
                    return 0
                lax.fori_loop(0, dpt // 16, zb, 0)
                pltpu.sync_copy(zbuf, deg_sh.at[pl.ds(s * dpt, dpt)])

        plsc.subcore_barrier()

        hdummy = h_r.at[pl.ds(0, _CH)]

        def compute_msg(par, prow):
            def grp(g, _):
                ew16 = ew_sv[prow, pl.ds(g * 16, 16)]
                for i in range(16):
                    w = ew16[i]
                    for j in range(nvec):
                        sl = pl.ds(j * 16, 16)
                        msg2[par, g * 16 + i, sl] = jnp.maximum(
                            gbuf2[par, g * 16 + i, sl]
                            + (wle_c[j] * w + blee_c[j]), epsv)
                return 0
            lax.fori_loop(0, _CH // 16, grp, 0)

        def fire_gather(par, row, gsem):
            pltpu.async_copy(h_r.at[gidx_all.at[row]], gbuf2.at[par], gsem)

        def seg_body(q, _):
            segrow = tile0 + q * _SEG
            pltpu.sync_copy(src_r.at[pl.ds(segrow, _SEG)], src_sv)
            qp = q % 2
            pltpu.sync_copy(dst_r.at[pl.ds(segrow, _SEG)], dst_sv.at[qp])
            pltpu.sync_copy(ew_r.at[pl.ds(segrow, _SEG)], ew_sv)
            goff = c * Nn

            def gx(i, _):
                for j in range(_CH // 16):
                    sl = pl.ds(j * 16, 16)
                    gidx_all[i, sl] = src_sv[i, sl] + goff
                return 0
            lax.fori_loop(0, _SEG, gx, 0)

            rel0 = q * _SEG


            def pair_body(pp, _):
                prow0 = 2 * pp
                prow1 = prow0 + 1
                rel_0 = rel0 + prow0

                @pl.when(rel_0 < ach)
                def _p(prow0=prow0, prow1=prow1):
                    d0 = pltpu.async_copy(h_r.at[gidx_all.at[prow0]],
                                          gbuf2.at[0], gsemA)
                    d1 = pltpu.async_copy(h_r.at[gidx_all.at[prow1]],
                                          gbuf2.at[1], gsemB)
                    d0.wait()
                    compute_msg(0, prow0)
                    pltpu.async_copy(msg2.at[0],
                                     agg_sh.at[dst_sv.at[qp, prow0]],
                                     ssemA, add=True).wait()
                    if compute_deg:
                        @pl.when(c == 0)
                        def _dg0():
                            pltpu.sync_copy(
                                ones_v, deg_sh.at[dst_sv.at[qp, prow0]],
                                add=True)
                    d1.wait()
                    compute_msg(1, prow1)
                    pltpu.async_copy(msg2.at[1],
                                     agg_sh.at[dst_sv.at[qp, prow1]],
                                     ssemB, add=True).wait()
                    if compute_deg:
                        @pl.when(c == 0)
                        def _dg1():
                            pltpu.sync_copy(
                                ones_v, deg_sh.at[dst_sv.at[qp, prow1]],
                                add=True)
                return 0
            lax.fori_loop(0, _SEG // 2, pair_body, 0)
            return 0
        lax.fori_loop(0, nseg, seg_body, 0)


        plsc.subcore_barrier()

        pltpu.sync_copy(agg_sh.at[pl.ds(s * spacing, spacing)],
                        agg_o.at[pl.ds(c * Nn + s * spacing, spacing)])
        if extra:
            @pl.when(s == _NS - 1)
            def _wx():
                pltpu.sync_copy(agg_sh.at[pl.ds(_NS * spacing, extra)],
                                agg_o.at[pl.ds(c * Nn + _NS * spacing, extra)])
        if compute_deg:
            @pl.when((c == 0) & (s < 5))
            def _wd():
                pltpu.sync_copy(deg_sh.at[pl.ds(s * dpt, dpt)], zbuf)
                pltpu.sync_copy(zbuf, deg_o.at[pl.ds(s * dpt, dpt)])

    return body(hsplit, src2, dst2, ew2, wle, blee)


def _mlp_body(agg_ref, h_ref, deg_ref, w1_ref, b1_ref, w2_ref, b2_ref, o_ref):
    hb = jnp.concatenate([h_ref[0], h_ref[1]], axis=1)
    ab = jnp.concatenate([agg_ref[0], agg_ref[1]], axis=1)
    deg = jnp.maximum(deg_ref[...], 1.0)
    out = ab / deg + hb
    t = jnp.maximum(
        jnp.dot(out, w1_ref[...], preferred_element_type=jnp.float32)
        + b1_ref[...], 0.0)
    hn = jnp.dot(t, w2_ref[...], preferred_element_type=jnp.float32) + b2_ref[...]
    hnew = jnp.maximum(hb + hn, 0.0)
    hh = hnew.shape[1] // 2
    o_ref[0] = hnew[:, :hh]
    o_ref[1] = hnew[:, hh:]


def _mlp(agg, h, deg, W1, b1, W2, b2):
    _, N, Hh = h.shape
    bn = N // 5
    H = 2 * Hh
    H2 = W1.shape[1]
    return pl.pallas_call(
        _mlp_body,
        grid=(N // bn,),
        in_specs=[
            pl.BlockSpec((2, bn, Hh), lambda i: (0, i, 0)),
            pl.BlockSpec((2, bn, Hh), lambda i: (0, i, 0)),
            pl.BlockSpec((bn, 1), lambda i: (i, 0)),
            pl.BlockSpec((H, H2), lambda i: (0, 0)),
            pl.BlockSpec((1, H2), lambda i: (0, 0)),
            pl.BlockSpec((H2, H), lambda i: (0, 0)),
            pl.BlockSpec((1, H), lambda i: (0, 0)),
        ],
        out_specs=pl.BlockSpec((2, bn, Hh), lambda i: (0, i, 0)),
        out_shape=jax.ShapeDtypeStruct((2, N, Hh), jnp.float32),
    )(agg, h, deg, W1, b1.reshape(1, H2), W2, b2.reshape(1, H))


def _pool_body(h_ref, o_ref):
    @pl.when(pl.program_id(0) == 0)
    def _init():
        o_ref[...] = jnp.zeros_like(o_ref)
    s0 = jnp.sum(h_ref[0], axis=0, keepdims=True)
    s1 = jnp.sum(h_ref[1], axis=0, keepdims=True)
    o_ref[...] += jnp.concatenate([s0, s1], axis=0)


def _pool(h):
    _, N, Hh = h.shape
    bn = N // 5
    return pl.pallas_call(
        _pool_body,
        grid=(N // bn,),
        in_specs=[pl.BlockSpec((2, bn, Hh), lambda i: (0, i, 0))],
        out_specs=pl.BlockSpec((2, Hh), lambda i: (0, 0)),
        out_shape=jax.ShapeDtypeStruct((2, Hh), jnp.float32),
    )(h)


def _dec_body(n_nodes, sums_ref, qt_ref, wt_ref, bt_ref, w1_ref, b1_ref,
              w2_ref, b2_ref, o_ref):
    pooled = jnp.concatenate([sums_ref[0:1, :], sums_ref[1:2, :]], axis=1)
    pooled = pooled * (1.0 / n_nodes)
    temb = jnp.maximum(
        jnp.dot(qt_ref[...], wt_ref[...], preferred_element_type=jnp.float32)
        + bt_ref[...], 0.0)
    comb = pooled + temb
    d = jnp.maximum(
        jnp.dot(comb, w1_ref[...], preferred_element_type=jnp.float32)
        + b1_ref[...], 0.0)
    o_ref[...] = jnp.dot(d, w2_ref[...], preferred_element_type=jnp.float32) \
        + b2_ref[...]


def _decoder(sums, qt, Wt, bt, W1, b1, W2, b2, n_nodes):
    B = qt.shape[0]
    H = Wt.shape[1]
    OUT = W2.shape[1]
    return pl.pallas_call(
        functools.partial(_dec_body, float(n_nodes)),
        out_shape=jax.ShapeDtypeStruct((B, OUT), jnp.float32),
    )(sums, qt, Wt, bt.reshape(1, H), W1, b1.reshape(1, H), W2,
      b2.reshape(1, OUT))


def kernel(x, edge_index, edge_attr, query_time, params):
    p = params
    N = x.shape[0]
    H = p["W_enc"].shape[1]
    Hh = H // 2
    E = edge_index.shape[1]
    L = p["Wm1"].shape[0]
    s_bn = 1.0 / jnp.sqrt(jnp.float32(1.0 + 1e-5))

    src = edge_index[0].astype(jnp.int32)
    dst = edge_index[1].astype(jnp.int32)

    hs = _encode(x, p["W_enc"], p["b_enc"])
    ew = _edge_weights(edge_attr, p["We1"], p["be1"], p["We2"],
                       p["be2"]).reshape(E)

    nch = E // _CH
    tpc = -(-nch // _NS)
    tpc = -(-tpc // _SEG) * _SEG
    npad = _NS * tpc
    src2 = jnp.pad(src.reshape(nch, _CH), ((0, npad - nch), (0, 0)))
    dst2 = jnp.pad(dst.reshape(nch, _CH), ((0, npad - nch), (0, 0)))
    ew2 = jnp.pad(ew.reshape(nch, _CH), ((0, npad - nch), (0, 0)))

    deg = None
    for l in range(L):
        wle = p["Wle"][l].reshape(H)
        blee = p["ble"][l] + _EPS
        res = _sc_aggregate(hs.reshape(2 * N, Hh), src2, dst2, ew2, wle, blee,
                            nch, compute_deg=(l == 0))
        if l == 0:
            agg, deg = res[0], res[1]
        else:
            agg = res[0]
        g1 = p["gamma"][l] * s_bn
        W1 = p["Wm1"][l] * g1[None, :]
        b1 = p["bm1"][l] * g1 + p["beta"][l]
        hs = _mlp(agg.reshape(2, N, Hh), hs, deg.reshape(N, 1), W1, b1,
                  p["Wm2"][l], p["bm2"][l])

    sums = _pool(hs)
    return _decoder(sums, query_time, p["Wt"], p["bt"], p["Wd1"], p["bd1"],
                    p["Wd2"], p["bd2"], N)


# speedup vs baseline: 3.5527x; 1.0102x over previous
import functools

import jax
import jax.numpy as jnp
from jax import lax
from jax.experimental import pallas as pl
from jax.experimental.pallas import tpu as pltpu
from jax.experimental.pallas import tpu_sc as plsc

_EPS = 1e-7
_NS = 16
_NC = 2
_CH = 80


def _enc_body(x_ref, w_ref, b_ref, o_ref):
    h = jnp.maximum(
        jnp.dot(x_ref[...], w_ref[...], preferred_element_type=jnp.float32)
        + b_ref[...], 0.0)
    hh = h.shape[1] // 2
    o_ref[0] = h[:, :hh]
    o_ref[1] = h[:, hh:]


def _encode(x, W, b):
    N, Din = x.shape
    bn = N // 5
    H = W.shape[1]
    return pl.pallas_call(
        _enc_body,
        grid=(N // bn,),
        in_specs=[
            pl.BlockSpec((bn, Din), lambda i: (i, 0)),
            pl.BlockSpec((Din, H), lambda i: (0, 0)),
            pl.BlockSpec((1, H), lambda i: (0, 0)),
        ],
        out_specs=pl.BlockSpec((2, bn, H // 2), lambda i: (0, i, 0)),
        out_shape=jax.ShapeDtypeStruct((2, N, H // 2), jnp.float32),
    )(x, W, b.reshape(1, H))


def _edge_body(a_ref, w1_ref, b1_ref, w2_ref, b2_ref, o_ref):
    t = jnp.maximum(
        jnp.dot(a_ref[...], w1_ref[...], preferred_element_type=jnp.float32)
        + b1_ref[...], 0.0)
    z = jnp.dot(t, w2_ref[...], preferred_element_type=jnp.float32) + b2_ref[...]
    o_ref[...] = 1.0 / (1.0 + jnp.exp(-z))


def _edge_weights(ea, W1, b1, W2, b2):
    E, De = ea.shape
    be = E // 160
    H = W1.shape[1]
    return pl.pallas_call(
        _edge_body,
        grid=(E // be,),
        in_specs=[
            pl.BlockSpec((be, De), lambda i: (i, 0)),
            pl.BlockSpec((De, H), lambda i: (0, 0)),
            pl.BlockSpec((1, H), lambda i: (0, 0)),
            pl.BlockSpec((H, 1), lambda i: (0, 0)),
            pl.BlockSpec((1, 1), lambda i: (0, 0)),
        ],
        out_specs=pl.BlockSpec((be, 1), lambda i: (i, 0)),
        out_shape=jax.ShapeDtypeStruct((E, 1), jnp.float32),
    )(ea, W1, b1.reshape(1, H), W2, b2.reshape(1, 1))


_SEG = 8

def _sc_aggregate(hsplit, src2, dst2, ew2, wle, blee, nch_real, compute_deg):
    TN, Hh = hsplit.shape
    Nn = TN // 2
    npad = src2.shape[0]
    nch = npad
    assert nch_real % 2 == 0
    tpc = npad // _NS
    nseg = tpc // _SEG
    spacing = (Nn // (_NS * 8)) * 8
    extra = Nn - _NS * spacing
    dpt = Nn // 5
    nvec = Hh // 16

    mesh = plsc.VectorSubcoreMesh(
        core_axis_name="c", subcore_axis_name="s",
        num_cores=_NC, num_subcores=_NS)

    out_type = [jax.ShapeDtypeStruct((TN, Hh), jnp.float32)]
    if compute_deg:
        out_type.append(jax.ShapeDtypeStruct((Nn,), jnp.float32))

    scratch = [
        pltpu.VMEM((_SEG, _CH), jnp.int32),
        pltpu.VMEM((_SEG, _CH), jnp.int32),
        pltpu.VMEM((_SEG, _CH), jnp.float32),
        pltpu.VMEM((2, _CH, Hh), jnp.float32),
        pltpu.VMEM((2, _CH, Hh), jnp.float32),
        pltpu.VMEM((Hh,), jnp.float32),
        pltpu.VMEM((Hh,), jnp.float32),
        pltpu.VMEM((_CH,), jnp.float32),
        pltpu.VMEM((dpt,), jnp.float32),
        pltpu.VMEM_SHARED((Nn, Hh), jnp.float32),
        pltpu.VMEM_SHARED((Nn,), jnp.float32),
        pltpu.SemaphoreType.DMA,
        pltpu.SemaphoreType.DMA,
        pltpu.SemaphoreType.DMA,
        pltpu.SemaphoreType.DMA,
    ]

    @functools.partial(pl.kernel, out_type=out_type, mesh=mesh,
                       scratch_types=scratch)
    def body(h_r, src_r, dst_r, ew_r, wle_r, blee_r, *rest):
        if compute_deg:
            agg_o, deg_o = rest[0], rest[1]
            sc = rest[2:]
        else:
            agg_o = rest[0]
            sc = rest[1:]
        (src_sv, dst_sv, ew_sv, gbuf2, msg2, wle_v, blee_v, ones_v,
         zbuf, agg_sh, deg_sh, gsemA, gsemB, ssemA, ssemB) = sc
        c = lax.axis_index("c")
        s = lax.axis_index("s")
        zeros16 = jnp.zeros((16,), jnp.float32)
        ones16 = jnp.full((16,), 1.0, jnp.float32)
        epsv = jnp.full((16,), _EPS, jnp.float32)
        ach = jnp.minimum(tpc, jnp.maximum(0, nch_real - s * tpc))
        tile0 = s * tpc

        pltpu.sync_copy(wle_r.at[pl.ds(c * Hh, Hh)], wle_v)
        pltpu.sync_copy(blee_r.at[pl.ds(c * Hh, Hh)], blee_v)
        wle_c = [wle_v[pl.ds(j * 16, 16)] for j in range(nvec)]
        blee_c = [blee_v[pl.ds(j * 16, 16)] for j in range(nvec)]

        def zrow(e, _):
            for j in range(nvec):
                msg2[0, e, pl.ds(j * 16, 16)] = zeros16
            return 0
        lax.fori_loop(0, _CH, zrow, 0)
        off = 0
        while off < spacing:
            step = min(_CH, spacing - off)
            pltpu.sync_copy(msg2.at[0].at[pl.ds(0, step)],
                            agg_sh.at[pl.ds(s * spacing + off, step)])
            off += step
        if extra:
            @pl.when(s == _NS - 1)
            def _zx():
                pltpu.sync_copy(msg2.at[0].at[pl.ds(0, extra)],
                                agg_sh.at[pl.ds(_NS * spacing, extra)])
        if compute_deg:
            def zo(i, _):
                ones_v[pl.ds(i * 16, 16)] = ones16
                return 0
            lax.fori_loop(0, _CH // 16, zo, 0)

            @pl.when((c == 0) & (s < 5))
            def _zd():
                def zb(i, _):
                    zbuf[pl.ds(i * 16, 16)] = zeros16
                    return 0
                lax.fori_loop(0, dpt // 16, zb, 0)
                pltpu.sync_copy(zbuf, deg_sh.at[pl.ds(s * dpt, dpt)])

        plsc.subcore_barrier()

        hdummy = h_r.at[pl.ds(0, _CH)]

        def compute_msg(par, prow):
            def grp(g, _):
                ew16 = ew_sv[prow, pl.ds(g * 16, 16)]
                for i in range(16):
                    w = ew16[i]
                    for j in range(nvec):
                        sl = pl.ds(j * 16, 16)
                        msg2[par, g * 16 + i, sl] = jnp.maximum(
                            gbuf2[par, g * 16 + i, sl]
                            + (wle_c[j] * w + blee_c[j]), epsv)
                return 0
            lax.fori_loop(0, _CH // 16, grp, 0)

        def seg_body(q, _):
            segrow = tile0 + q * _SEG
            pltpu.sync_copy(src_r.at[pl.ds(segrow, _SEG)], src_sv)
            pltpu.sync_copy(dst_r.at[pl.ds(segrow, _SEG)], dst_sv)
            pltpu.sync_copy(ew_r.at[pl.ds(segrow, _SEG)], ew_sv)
            goff = c * Nn

            def gx(i, _):
                for j in range(_CH // 16):
                    sl = pl.ds(j * 16, 16)
                    src_sv[i, sl] = src_sv[i, sl] + goff
                return 0
            lax.fori_loop(0, _SEG, gx, 0)

            rel0 = q * _SEG


            def pair_body(pp, _):
                prow0 = 2 * pp
                prow1 = prow0 + 1
                rel_0 = rel0 + prow0

                @pl.when(rel_0 < ach)
                def _p(prow0=prow0, prow1=prow1):
                    d0 = pltpu.async_copy(h_r.at[src_sv.at[prow0]],
                                          gbuf2.at[0], gsemA)
                    d1 = pltpu.async_copy(h_r.at[src_sv.at[prow1]],
                                          gbuf2.at[1], gsemB)
                    d0.wait()
                    compute_msg(0, prow0)
                    pltpu.async_copy(msg2.at[0],
                                     agg_sh.at[dst_sv.at[prow0]],
                                     ssemA, add=True).wait()
                    if compute_deg:
                        @pl.when(c == 0)
                        def _dg0():
                            pltpu.sync_copy(
                                ones_v, deg_sh.at[dst_sv.at[prow0]],
                                add=True)
                    d1.wait()
                    compute_msg(1, prow1)
                    pltpu.async_copy(msg2.at[1],
                                     agg_sh.at[dst_sv.at[prow1]],
                                     ssemB, add=True).wait()
                    if compute_deg:
                        @pl.when(c == 0)
                        def _dg1():
                            pltpu.sync_copy(
                                ones_v, deg_sh.at[dst_sv.at[prow1]],
                                add=True)
                return 0
            lax.fori_loop(0, _SEG // 2, pair_body, 0)
            return 0
        lax.fori_loop(0, nseg, seg_body, 0)


        plsc.subcore_barrier()

        pltpu.sync_copy(agg_sh.at[pl.ds(s * spacing, spacing)],
                        agg_o.at[pl.ds(c * Nn + s * spacing, spacing)])
        if extra:
            @pl.when(s == _NS - 1)
            def _wx():
                pltpu.sync_copy(agg_sh.at[pl.ds(_NS * spacing, extra)],
                                agg_o.at[pl.ds(c * Nn + _NS * spacing, extra)])
        if compute_deg:
            @pl.when((c == 0) & (s < 5))
            def _wd():
                pltpu.sync_copy(deg_sh.at[pl.ds(s * dpt, dpt)], zbuf)
                pltpu.sync_copy(zbuf, deg_o.at[pl.ds(s * dpt, dpt)])

    return body(hsplit, src2, dst2, ew2, wle, blee)


def _mlp_body(agg_ref, h_ref, deg_ref, w1_ref, b1_ref, w2_ref, b2_ref, o_ref):
    hb = jnp.concatenate([h_ref[0], h_ref[1]], axis=1)
    ab = jnp.concatenate([agg_ref[0], agg_ref[1]], axis=1)
    deg = jnp.maximum(deg_ref[...], 1.0)
    out = ab / deg + hb
    t = jnp.maximum(
        jnp.dot(out, w1_ref[...], preferred_element_type=jnp.float32)
        + b1_ref[...], 0.0)
    hn = jnp.dot(t, w2_ref[...], preferred_element_type=jnp.float32) + b2_ref[...]
    hnew = jnp.maximum(hb + hn, 0.0)
    hh = hnew.shape[1] // 2
    o_ref[0] = hnew[:, :hh]
    o_ref[1] = hnew[:, hh:]


def _mlp(agg, h, deg, W1, b1, W2, b2):
    _, N, Hh = h.shape
    bn = N // 5
    H = 2 * Hh
    H2 = W1.shape[1]
    return pl.pallas_call(
        _mlp_body,
        grid=(N // bn,),
        in_specs=[
            pl.BlockSpec((2, bn, Hh), lambda i: (0, i, 0)),
            pl.BlockSpec((2, bn, Hh), lambda i: (0, i, 0)),
            pl.BlockSpec((bn, 1), lambda i: (i, 0)),
            pl.BlockSpec((H, H2), lambda i: (0, 0)),
            pl.BlockSpec((1, H2), lambda i: (0, 0)),
            pl.BlockSpec((H2, H), lambda i: (0, 0)),
            pl.BlockSpec((1, H), lambda i: (0, 0)),
        ],
        out_specs=pl.BlockSpec((2, bn, Hh), lambda i: (0, i, 0)),
        out_shape=jax.ShapeDtypeStruct((2, N, Hh), jnp.float32),
    )(agg, h, deg, W1, b1.reshape(1, H2), W2, b2.reshape(1, H))


def _pool_body(h_ref, o_ref):
    @pl.when(pl.program_id(0) == 0)
    def _init():
        o_ref[...] = jnp.zeros_like(o_ref)
    s0 = jnp.sum(h_ref[0], axis=0, keepdims=True)
    s1 = jnp.sum(h_ref[1], axis=0, keepdims=True)
    o_ref[...] += jnp.concatenate([s0, s1], axis=0)


def _pool(h):
    _, N, Hh = h.shape
    bn = N // 5
    return pl.pallas_call(
        _pool_body,
        grid=(N // bn,),
        in_specs=[pl.BlockSpec((2, bn, Hh), lambda i: (0, i, 0))],
        out_specs=pl.BlockSpec((2, Hh), lambda i: (0, 0)),
        out_shape=jax.ShapeDtypeStruct((2, Hh), jnp.float32),
    )(h)


def _dec_body(n_nodes, sums_ref, qt_ref, wt_ref, bt_ref, w1_ref, b1_ref,
              w2_ref, b2_ref, o_ref):
    pooled = jnp.concatenate([sums_ref[0:1, :], sums_ref[1:2, :]], axis=1)
    pooled = pooled * (1.0 / n_nodes)
    temb = jnp.maximum(
        jnp.dot(qt_ref[...], wt_ref[...], preferred_element_type=jnp.float32)
        + bt_ref[...], 0.0)
    comb = pooled + temb
    d = jnp.maximum(
        jnp.dot(comb, w1_ref[...], preferred_element_type=jnp.float32)
        + b1_ref[...], 0.0)
    o_ref[...] = jnp.dot(d, w2_ref[...], preferred_element_type=jnp.float32) \
        + b2_ref[...]


def _decoder(sums, qt, Wt, bt, W1, b1, W2, b2, n_nodes):
    B = qt.shape[0]
    H = Wt.shape[1]
    OUT = W2.shape[1]
    return pl.pallas_call(
        functools.partial(_dec_body, float(n_nodes)),
        out_shape=jax.ShapeDtypeStruct((B, OUT), jnp.float32),
    )(sums, qt, Wt, bt.reshape(1, H), W1, b1.reshape(1, H), W2,
      b2.reshape(1, OUT))


def kernel(x, edge_index, edge_attr, query_time, params):
    p = params
    N = x.shape[0]
    H = p["W_enc"].shape[1]
    Hh = H // 2
    E = edge_index.shape[1]
    L = p["Wm1"].shape[0]
    s_bn = 1.0 / jnp.sqrt(jnp.float32(1.0 + 1e-5))

    src = edge_index[0].astype(jnp.int32)
    dst = edge_index[1].astype(jnp.int32)

    hs = _encode(x, p["W_enc"], p["b_enc"])
    ew = _edge_weights(edge_attr, p["We1"], p["be1"], p["We2"],
                       p["be2"]).reshape(E)

    nch = E // _CH
    tpc = -(-nch // _NS)
    tpc = -(-tpc // _SEG) * _SEG
    npad = _NS * tpc
    src2 = jnp.pad(src.reshape(nch, _CH), ((0, npad - nch), (0, 0)))
    dst2 = jnp.pad(dst.reshape(nch, _CH), ((0, npad - nch), (0, 0)))
    ew2 = jnp.pad(ew.reshape(nch, _CH), ((0, npad - nch), (0, 0)))

    deg = None
    for l in range(L):
        wle = p["Wle"][l].reshape(H)
        blee = p["ble"][l] + _EPS
        res = _sc_aggregate(hs.reshape(2 * N, Hh), src2, dst2, ew2, wle, blee,
                            nch, compute_deg=(l == 0))
        if l == 0:
            agg, deg = res[0], res[1]
        else:
            agg = res[0]
        g1 = p["gamma"][l] * s_bn
        W1 = p["Wm1"][l] * g1[None, :]
        b1 = p["bm1"][l] * g1 + p["beta"][l]
        hs = _mlp(agg.reshape(2, N, Hh), hs, deg.reshape(N, 1), W1, b1,
                  p["Wm2"][l], p["bm2"][l])

    sums = _pool(hs)
    return _decoder(sums, query_time, p["Wt"], p["bt"], p["Wd1"], p["bd1"],
                    p["Wd2"], p["bd2"], N)
